# baseline (speedup 1.0000x reference)
import functools

import jax
import jax.numpy as jnp
from jax import lax
from jax.experimental import pallas as pl
from jax.experimental.pallas import tpu as pltpu
from jax.experimental.pallas import tpu_sc as plsc

_NC = 2
_NS = 16
_NW = _NC * _NS
_CH = 64
_L = 16


@functools.lru_cache(maxsize=None)
def _make_decode(K, D, N):
    per_w = N // _NW
    n_ch = per_w // _CH
    mesh = plsc.VectorSubcoreMesh(core_axis_name="c", subcore_axis_name="s")

    @functools.partial(
        pl.kernel,
        mesh=mesh,
        out_type=jax.ShapeDtypeStruct((N, D), jnp.float32),
        scratch_types=[
            pltpu.VMEM((n_ch, _CH), jnp.int32),
            pltpu.VMEM((per_w,), jnp.float32),
            pltpu.VMEM((per_w * _L,), jnp.float32),
            pltpu.VMEM((_CH, D), jnp.float32),
            pltpu.VMEM((_CH, D), jnp.float32),
            pltpu.VMEM((_CH, D), jnp.float32),
            pltpu.VMEM((_CH, D), jnp.float32),
            pltpu.SemaphoreType.DMA,
            pltpu.SemaphoreType.DMA,
            pltpu.SemaphoreType.DMA,
            pltpu.SemaphoreType.DMA,
            pltpu.SemaphoreType.DMA,
            pltpu.SemaphoreType.DMA,
            pltpu.SemaphoreType.DMA,
            pltpu.SemaphoreType.DMA,
            pltpu.SemaphoreType.DMA,
        ],
    )
    def decode_k(emb_hbm, usage_hbm, idx_hbm, out_hbm,
                 idx_v, recip_v, rsp_v, rows_a, rows_b, rows_c, rows_d,
                 gsem_a, gsem_b, gsem_c, gsem_d,
                 osem_a, osem_b, osem_c, osem_d, usem):
        wid = lax.axis_index("s") * _NC + lax.axis_index("c")
        base = wid * per_w
        bufs = (rows_a, rows_b, rows_c, rows_d)
        gsems = (gsem_a, gsem_b, gsem_c, gsem_d)
        osems = (osem_a, osem_b, osem_c, osem_d)
        nb = len(bufs)

        pltpu.sync_copy(idx_hbm.at[wid], idx_v)

        def start_gather(c):
            b = c % nb
            return pltpu.async_copy(emb_hbm.at[idx_v.at[c]], bufs[b], gsems[b])

        gathers = {0: start_gather(0)}

        ucopies = [
            pltpu.async_copy(usage_hbm.at[idx_v.at[c]],
                             recip_v.at[pl.ds(c * _CH, _CH)], usem)
            for c in range(n_ch)
        ]
        for cp in ucopies:
            cp.wait()

        def splat_body(g, carry):
            r16 = 1.0 / recip_v[pl.ds(g * _L, _L)]
            for rr in range(_L):
                rsp_v[pl.ds((g * _L + rr) * _L, _L)] = jnp.full(
                    (_L,), r16[rr], jnp.float32)
            return carry

        lax.fori_loop(0, per_w // _L, splat_body, 0)

        def scale_chunk(c):
            buf = bufs[c % nb]

            def srow(r, carry):
                s = rsp_v[pl.ds((c * _CH + r) * _L, _L)]
                for j in range(D // _L):
                    buf[r, pl.ds(j * _L, _L)] = buf[r, pl.ds(j * _L, _L)] * s
                return carry

            lax.fori_loop(0, _CH, srow, 0)

        outs = {}
        for c in range(n_ch):
            b = c % nb
            gathers[c].wait()
            if c + 1 < n_ch:
                if c + 1 - nb >= 0:
                    outs[c + 1 - nb].wait()
                gathers[c + 1] = start_gather(c + 1)
            scale_chunk(c)
            outs[c] = pltpu.async_copy(
                bufs[b], out_hbm.at[pl.ds(base + c * _CH, _CH)], osems[b])
        for c in range(max(0, n_ch - nb), n_ch):
            outs[c].wait()

    return decode_k


def kernel(indices, embedding_sum, cluster_usage):
    K, D = embedding_sum.shape
    B, T = indices.shape
    N = B * T
    idx3 = indices.reshape(_NW, N // _NW // _CH, _CH).astype(jnp.int32)
    out = _make_decode(K, D, N)(embedding_sum, cluster_usage, idx3)
    return out.reshape(B, T, D)


# speedup vs baseline: 1.0956x; 1.0956x over previous
import functools

import jax
import jax.numpy as jnp
from jax import lax
from jax.experimental import pallas as pl
from jax.experimental.pallas import tpu as pltpu
from jax.experimental.pallas import tpu_sc as plsc

_NC = 2
_NS = 16
_NW = _NC * _NS
_CH = 128
_L = 16


@functools.lru_cache(maxsize=None)
def _make_decode(K, D, N):
    per_w = N // _NW
    n_ch = per_w // _CH
    mesh = plsc.VectorSubcoreMesh(core_axis_name="c", subcore_axis_name="s")

    @functools.partial(
        pl.kernel,
        mesh=mesh,
        out_type=jax.ShapeDtypeStruct((N, D), jnp.float32),
        scratch_types=[
            pltpu.VMEM((n_ch, _CH), jnp.int32),
            pltpu.VMEM((per_w,), jnp.float32),
            pltpu.VMEM((per_w * _L,), jnp.float32),
            pltpu.VMEM((_CH, D), jnp.float32),
            pltpu.VMEM((_CH, D), jnp.float32),
            pltpu.VMEM((_CH, D), jnp.float32),
            pltpu.SemaphoreType.DMA,
            pltpu.SemaphoreType.DMA,
            pltpu.SemaphoreType.DMA,
            pltpu.SemaphoreType.DMA,
            pltpu.SemaphoreType.DMA,
            pltpu.SemaphoreType.DMA,
            pltpu.SemaphoreType.DMA,
        ],
    )
    def decode_k(emb_hbm, usage_hbm, idx_hbm, out_hbm,
                 idx_v, recip_v, rsp_v, rows_a, rows_b, rows_c,
                 gsem_a, gsem_b, gsem_c, osem_a, osem_b, osem_c, usem):
        wid = lax.axis_index("s") * _NC + lax.axis_index("c")
        base = wid * per_w
        bufs = (rows_a, rows_b, rows_c)
        gsems = (gsem_a, gsem_b, gsem_c)
        osems = (osem_a, osem_b, osem_c)
        nb = len(bufs)

        pltpu.sync_copy(idx_hbm.at[wid], idx_v)

        def start_gather(c):
            b = c % nb
            return pltpu.async_copy(emb_hbm.at[idx_v.at[c]], bufs[b], gsems[b])

        gathers = {0: start_gather(0), 1: start_gather(1)}

        ucopies = [
            pltpu.async_copy(usage_hbm.at[idx_v.at[c]],
                             recip_v.at[pl.ds(c * _CH, _CH)], usem)
            for c in range(n_ch)
        ]
        for cp in ucopies:
            cp.wait()

        def splat_body(g, carry):
            r16 = 1.0 / recip_v[pl.ds(g * _L, _L)]
            for rr in range(_L):
                rsp_v[pl.ds((g * _L + rr) * _L, _L)] = jnp.full(
                    (_L,), r16[rr], jnp.float32)
            return carry

        lax.fori_loop(0, per_w // _L, splat_body, 0)

        def scale_chunk(c):
            buf = bufs[c % nb]

            def srow(r, carry):
                s = rsp_v[pl.ds((c * _CH + r) * _L, _L)]
                for j in range(D // _L):
                    buf[r, pl.ds(j * _L, _L)] = buf[r, pl.ds(j * _L, _L)] * s
                return carry

            lax.fori_loop(0, _CH, srow, 0)

        outs = {}
        for c in range(n_ch):
            b = c % nb
            gathers[c].wait()
            if c + 2 < n_ch:
                if c + 2 - nb >= 0:
                    outs[c + 2 - nb].wait()
                gathers[c + 2] = start_gather(c + 2)
            scale_chunk(c)
            outs[c] = pltpu.async_copy(
                bufs[b], out_hbm.at[pl.ds(base + c * _CH, _CH)], osems[b])
        for c in range(max(0, n_ch - nb), n_ch):
            outs[c].wait()

    return decode_k


def kernel(indices, embedding_sum, cluster_usage):
    K, D = embedding_sum.shape
    B, T = indices.shape
    N = B * T
    idx3 = indices.reshape(_NW, N // _NW // _CH, _CH).astype(jnp.int32)
    out = _make_decode(K, D, N)(embedding_sum, cluster_usage, idx3)
    return out.reshape(B, T, D)
